# baseline (speedup 1.0000x reference)
import functools

import jax
import jax.numpy as jnp
from jax import lax
from jax.experimental import pallas as pl
from jax.experimental.pallas import tpu as pltpu
from jax.experimental.pallas import tpu_sc as plsc

NC = 2
NS = 16
NW = NC * NS


def _sc_hist(n_pad, n_edges, window):
  epw = n_edges // NW
  nwin = epw // window
  chunk = n_pad // NS
  mesh = plsc.VectorSubcoreMesh(core_axis_name="c", subcore_axis_name="s")

  def body(dst_hbm, zeros_hbm, ones_hbm, out_hbm, acc_sh, ones_v,
           didx0, didx1, isem0, isem1, ssem0, ssem1):
    didx = (didx0, didx1)
    isem = (isem0, isem1)
    ssem = (ssem0, ssem1)
    c = lax.axis_index("c")
    s = lax.axis_index("s")
    wid = c * NS + s
    base = wid * epw
    row = pl.ds(s * chunk, chunk)

    pltpu.sync_copy(zeros_hbm.at[row], acc_sh.at[row])
    pltpu.sync_copy(ones_hbm, ones_v)
    plsc.subcore_barrier()

    def load_idx(p, off):
      pltpu.async_copy(dst_hbm.at[pl.ds(off, window)], didx[p], isem[p])

    def wait_idx(p):
      pltpu.make_async_copy(
          dst_hbm.at[pl.ds(0, window)], didx[p], isem[p]).wait()

    def wait_scat(p):
      pltpu.make_async_copy(ones_v, acc_sh.at[didx[p]], ssem[p]).wait()

    load_idx(0, base)
    load_idx(1, base + window)

    def step(j, carry):
      for p in (0, 1):
        off = base + (2 * j + p) * window

        @pl.when(j > 0)
        def _():
          wait_scat(p)
          load_idx(p, off)

        wait_idx(p)
        pltpu.async_copy(ones_v, acc_sh.at[didx[p]], ssem[p], add=True)
      return carry

    lax.fori_loop(0, nwin // 2, step, 0)
    wait_scat(0)
    wait_scat(1)
    plsc.subcore_barrier()
    pltpu.sync_copy(acc_sh.at[row], out_hbm.at[c, row])

  return pl.kernel(
      body,
      out_type=jax.ShapeDtypeStruct((NC, n_pad), jnp.float32),
      mesh=mesh,
      compiler_params=pltpu.CompilerParams(use_tc_tiling_on_sc=False),
      scratch_types=[
          pltpu.VMEM_SHARED((n_pad,), jnp.float32),
          pltpu.VMEM((window,), jnp.float32),
          pltpu.VMEM((window,), jnp.int32),
          pltpu.VMEM((window,), jnp.int32),
          pltpu.SemaphoreType.DMA,
          pltpu.SemaphoreType.DMA,
          pltpu.SemaphoreType.DMA,
          pltpu.SemaphoreType.DMA,
      ],
  )


def _sc_gs(n_pad, n_edges, window, fp):
  epw = n_edges // NW
  nwin = epw // window
  chunk = n_pad // NS
  shp = lambda m: (m, fp) if fp else (m,)
  mesh = plsc.VectorSubcoreMesh(core_axis_name="c", subcore_axis_name="s")

  def body(src_hbm, dst_hbm, zeros_hbm, tab_hbm, out_hbm,
           tab_sh, acc_sh, val0, val1, sidx0, sidx1, didx0, didx1,
           isem0, isem1, gsem0, gsem1, ssem0, ssem1):
    val = (val0, val1)
    sidx = (sidx0, sidx1)
    didx = (didx0, didx1)
    isem = (isem0, isem1)
    gsem = (gsem0, gsem1)
    ssem = (ssem0, ssem1)
    c = lax.axis_index("c")
    s = lax.axis_index("s")
    wid = c * NS + s
    base = wid * epw
    row = pl.ds(s * chunk, chunk)

    pltpu.sync_copy(tab_hbm.at[row], tab_sh.at[row])
    pltpu.sync_copy(zeros_hbm.at[row], acc_sh.at[row])
    plsc.subcore_barrier()

    def load_idx(p, off):
      pltpu.async_copy(src_hbm.at[pl.ds(off, window)], sidx[p], isem[p])
      pltpu.async_copy(dst_hbm.at[pl.ds(off, window)], didx[p], isem[p])

    def wait_idx(p):
      pltpu.make_async_copy(
          src_hbm.at[pl.ds(0, window)], sidx[p], isem[p]).wait()
      pltpu.make_async_copy(
          dst_hbm.at[pl.ds(0, window)], didx[p], isem[p]).wait()

    def wait_scat(p):
      pltpu.make_async_copy(val[p], acc_sh.at[didx[p]], ssem[p]).wait()

    load_idx(0, base)
    load_idx(1, base + window)

    def step(j, carry):
      for p in (0, 1):
        off = base + (2 * j + p) * window

        @pl.when(j > 0)
        def _():
          wait_scat(p)
          load_idx(p, off)

        wait_idx(p)
        pltpu.async_copy(tab_sh.at[sidx[p]], val[p], gsem[p]).wait()
        pltpu.async_copy(val[p], acc_sh.at[didx[p]], ssem[p], add=True)
      return carry

    lax.fori_loop(0, nwin // 2, step, 0)
    wait_scat(0)
    wait_scat(1)
    plsc.subcore_barrier()
    pltpu.sync_copy(acc_sh.at[row], out_hbm.at[c, row])

  return pl.kernel(
      body,
      out_type=jax.ShapeDtypeStruct((NC,) + shp(n_pad), jnp.float32),
      mesh=mesh,
      compiler_params=pltpu.CompilerParams(use_tc_tiling_on_sc=False),
      scratch_types=[
          pltpu.VMEM_SHARED(shp(n_pad), jnp.float32),
          pltpu.VMEM_SHARED(shp(n_pad), jnp.float32),
          pltpu.VMEM(shp(window), jnp.float32),
          pltpu.VMEM(shp(window), jnp.float32),
          pltpu.VMEM((window,), jnp.int32),
          pltpu.VMEM((window,), jnp.int32),
          pltpu.VMEM((window,), jnp.int32),
          pltpu.VMEM((window,), jnp.int32),
          pltpu.SemaphoreType.DMA,
          pltpu.SemaphoreType.DMA,
          pltpu.SemaphoreType.DMA,
          pltpu.SemaphoreType.DMA,
          pltpu.SemaphoreType.DMA,
          pltpu.SemaphoreType.DMA,
      ],
  )


_HI = lax.Precision.HIGHEST


def _tc_node1(degp_ref, x_ref, rep_ref, dis_ref, y1_ref):
  dis8 = lax.rsqrt(1.0 + degp_ref[0] + degp_ref[1])
  dis_ref[...] = dis8
  x_r = x_ref[...].astype(jnp.bfloat16).astype(jnp.float32)
  disr = jnp.dot(dis8, rep_ref[...], precision=_HI)
  y1_ref[...] = x_r * disr


def _tc_node2(accp_ref, y1_ref, dis_ref, rep_ref, b_ref, b1_ref, c_ref,
              y2_ref):
  dis8 = dis_ref[...]
  disr = jnp.dot(dis8, rep_ref[...], precision=_HI)
  p1 = disr * (accp_ref[0] + accp_ref[1] + y1_ref[...])
  h = jnp.dot(p1, b_ref[...], precision=_HI) + b1_ref[...]
  h = jnp.maximum(h, 0.0)
  h = h.astype(jnp.bfloat16).astype(jnp.float32)
  z = jnp.dot(h, c_ref[...], precision=_HI)
  y2_ref[...] = z * dis8


def _tc_node3(accp_ref, y2_ref, dis_ref, b2_ref, out_ref):
  out_ref[...] = dis_ref[...] * (accp_ref[0] + accp_ref[1] + y2_ref[...]) \
      + b2_ref[0]


def kernel(x, edge_index, W1, b1, W2, b2):
  n = x.shape[0]
  nf_in = x.shape[1]
  nf_hid = W1.shape[1]
  n_edges = edge_index.shape[1]
  window = 4000
  window_h = 2000
  window_r = 1000
  n_pad = 102400
  rows = n_pad // 128
  fp = 8
  g = n_pad // fp

  src = edge_index[0].astype(jnp.int32)
  dst = edge_index[1].astype(jnp.int32)
  W1r = jnp.zeros((fp, nf_hid), jnp.float32).at[:nf_in].set(
      W1.astype(jnp.bfloat16).astype(jnp.float32))
  W2r = W2.astype(jnp.bfloat16).astype(jnp.float32)
  eye8 = jnp.eye(fp, dtype=jnp.float32)
  rep_m = jnp.kron(eye8, jnp.ones((1, fp), jnp.float32))
  b_m = jnp.kron(eye8, W1r)
  c_m = jnp.kron(eye8, W2r)
  b1_t = jnp.tile(b1, fp).reshape(1, fp * nf_hid)
  zeros_n = jnp.zeros((n_pad,), jnp.float32)
  ones_w = jnp.ones((window_h,), jnp.float32)
  x_f = jnp.pad(x, ((0, n_pad - n), (0, fp - nf_in))).reshape(g, fp * fp)

  degp = _sc_hist(n_pad, n_edges, window_h)(dst, zeros_n, ones_w)

  gb = g // 4
  dis8, y1_f = pl.pallas_call(
      _tc_node1,
      grid=(4,),
      in_specs=[
          pl.BlockSpec((NC, gb, fp), lambda i: (0, i, 0)),
          pl.BlockSpec((gb, fp * fp), lambda i: (i, 0)),
          pl.BlockSpec((fp, fp * fp), lambda i: (0, 0)),
      ],
      out_specs=(
          pl.BlockSpec((gb, fp), lambda i: (i, 0)),
          pl.BlockSpec((gb, fp * fp), lambda i: (i, 0)),
      ),
      out_shape=(
          jax.ShapeDtypeStruct((g, fp), jnp.float32),
          jax.ShapeDtypeStruct((g, fp * fp), jnp.float32),
      ),
  )(degp.reshape(NC, g, fp), x_f, rep_m)

  zeros_nf = jnp.zeros((n_pad, fp), jnp.float32)
  acc1_rm = _sc_gs(n_pad, n_edges, window_r, fp)(
      src, dst, zeros_nf, y1_f.reshape(n_pad, fp))

  y2 = pl.pallas_call(
      _tc_node2,
      grid=(4,),
      in_specs=[
          pl.BlockSpec((NC, gb, fp * fp), lambda i: (0, i, 0)),
          pl.BlockSpec((gb, fp * fp), lambda i: (i, 0)),
          pl.BlockSpec((gb, fp), lambda i: (i, 0)),
          pl.BlockSpec((fp, fp * fp), lambda i: (0, 0)),
          pl.BlockSpec((fp * fp, fp * nf_hid), lambda i: (0, 0)),
          pl.BlockSpec((1, fp * nf_hid), lambda i: (0, 0)),
          pl.BlockSpec((fp * nf_hid, fp), lambda i: (0, 0)),
      ],
      out_specs=pl.BlockSpec((gb, fp), lambda i: (i, 0)),
      out_shape=jax.ShapeDtypeStruct((g, fp), jnp.float32),
  )(acc1_rm.reshape(NC, g, fp * fp), y1_f, dis8, rep_m, b_m, b1_t, c_m)

  acc2 = _sc_gs(n_pad, n_edges, window, 0)(
      src, dst, zeros_n, y2.reshape(n_pad))

  out = pl.pallas_call(
      _tc_node3,
      in_specs=[
          pl.BlockSpec(memory_space=pltpu.MemorySpace.VMEM),
          pl.BlockSpec(memory_space=pltpu.MemorySpace.VMEM),
          pl.BlockSpec(memory_space=pltpu.MemorySpace.VMEM),
          pl.BlockSpec(memory_space=pltpu.SMEM),
      ],
      out_shape=jax.ShapeDtypeStruct((rows, 128), jnp.float32),
  )(acc2.reshape(NC, rows, 128), y2.reshape(rows, 128), dis8.reshape(rows, 128), b2)

  return out.reshape(n_pad)[:n].reshape(n, 1)


# speedup vs baseline: 1.0248x; 1.0248x over previous
import functools

import jax
import jax.numpy as jnp
from jax import lax
from jax.experimental import pallas as pl
from jax.experimental.pallas import tpu as pltpu
from jax.experimental.pallas import tpu_sc as plsc

NC = 2
NS = 16
NW = NC * NS


def _sc_hist(n_pad, n_edges, window):
  epw = n_edges // NW
  nwin = epw // window
  chunk = n_pad // NS
  mesh = plsc.VectorSubcoreMesh(core_axis_name="c", subcore_axis_name="s")

  def body(dst_hbm, zeros_hbm, ones_hbm, out_hbm, acc_sh, ones_v,
           didx0, didx1, isem0, isem1, ssem0, ssem1):
    didx = (didx0, didx1)
    isem = (isem0, isem1)
    ssem = (ssem0, ssem1)
    c = lax.axis_index("c")
    s = lax.axis_index("s")
    wid = c * NS + s
    base = wid * epw
    row = pl.ds(s * chunk, chunk)

    pltpu.sync_copy(zeros_hbm.at[row], acc_sh.at[row])
    pltpu.sync_copy(ones_hbm, ones_v)
    plsc.subcore_barrier()

    def load_idx(p, off):
      pltpu.async_copy(dst_hbm.at[pl.ds(off, window)], didx[p], isem[p])

    def wait_idx(p):
      pltpu.make_async_copy(
          dst_hbm.at[pl.ds(0, window)], didx[p], isem[p]).wait()

    def wait_scat(p):
      pltpu.make_async_copy(ones_v, acc_sh.at[didx[p]], ssem[p]).wait()

    load_idx(0, base)
    load_idx(1, base + window)

    def step(j, carry):
      for p in (0, 1):
        off = base + (2 * j + p) * window

        @pl.when(j > 0)
        def _():
          wait_scat(p)
          load_idx(p, off)

        wait_idx(p)
        pltpu.async_copy(ones_v, acc_sh.at[didx[p]], ssem[p], add=True)
      return carry

    lax.fori_loop(0, nwin // 2, step, 0)
    wait_scat(0)
    wait_scat(1)
    plsc.subcore_barrier()
    pltpu.sync_copy(acc_sh.at[row], out_hbm.at[c, row])

  return pl.kernel(
      body,
      out_type=jax.ShapeDtypeStruct((NC, n_pad), jnp.float32),
      mesh=mesh,
      compiler_params=pltpu.CompilerParams(use_tc_tiling_on_sc=False),
      scratch_types=[
          pltpu.VMEM_SHARED((n_pad,), jnp.float32),
          pltpu.VMEM((window,), jnp.float32),
          pltpu.VMEM((window,), jnp.int32),
          pltpu.VMEM((window,), jnp.int32),
          pltpu.SemaphoreType.DMA,
          pltpu.SemaphoreType.DMA,
          pltpu.SemaphoreType.DMA,
          pltpu.SemaphoreType.DMA,
      ],
  )


def _sc_gs(n_pad, n_edges, window, fp):
  epw = n_edges // NW
  nwin = epw // window
  chunk = n_pad // NS
  shp = lambda m: (m, fp) if fp else (m,)
  mesh = plsc.VectorSubcoreMesh(core_axis_name="c", subcore_axis_name="s")

  def body(src_hbm, dst_hbm, zeros_hbm, tab_hbm, out_hbm,
           tab_sh, acc_sh, val0, val1, sidx0, sidx1, didx0, didx1,
           isem0, isem1, gsem0, gsem1, ssem0, ssem1):
    val = (val0, val1)
    sidx = (sidx0, sidx1)
    didx = (didx0, didx1)
    isem = (isem0, isem1)
    gsem = (gsem0, gsem1)
    ssem = (ssem0, ssem1)
    c = lax.axis_index("c")
    s = lax.axis_index("s")
    wid = c * NS + s
    base = wid * epw
    row = pl.ds(s * chunk, chunk)

    pltpu.sync_copy(tab_hbm.at[row], tab_sh.at[row])
    pltpu.sync_copy(zeros_hbm.at[row], acc_sh.at[row])
    plsc.subcore_barrier()

    def load_idx(p, off):
      pltpu.async_copy(src_hbm.at[pl.ds(off, window)], sidx[p], isem[p])
      pltpu.async_copy(dst_hbm.at[pl.ds(off, window)], didx[p], isem[p])

    def wait_idx(p):
      pltpu.make_async_copy(
          src_hbm.at[pl.ds(0, window)], sidx[p], isem[p]).wait()
      pltpu.make_async_copy(
          dst_hbm.at[pl.ds(0, window)], didx[p], isem[p]).wait()

    def wait_scat(p):
      pltpu.make_async_copy(val[p], acc_sh.at[didx[p]], ssem[p]).wait()

    load_idx(0, base)
    load_idx(1, base + window)

    def step(j, carry):
      for p in (0, 1):
        off = base + (2 * j + p) * window

        @pl.when(j > 0)
        def _():
          wait_scat(p)
          load_idx(p, off)

        wait_idx(p)
        pltpu.async_copy(tab_sh.at[sidx[p]], val[p], gsem[p]).wait()
        pltpu.async_copy(val[p], acc_sh.at[didx[p]], ssem[p], add=True)
      return carry

    lax.fori_loop(0, nwin // 2, step, 0)
    wait_scat(0)
    wait_scat(1)
    plsc.subcore_barrier()
    pltpu.sync_copy(acc_sh.at[row], out_hbm.at[c, row])

  return pl.kernel(
      body,
      out_type=jax.ShapeDtypeStruct((NC,) + shp(n_pad), jnp.float32),
      mesh=mesh,
      compiler_params=pltpu.CompilerParams(use_tc_tiling_on_sc=False),
      scratch_types=[
          pltpu.VMEM_SHARED(shp(n_pad), jnp.float32),
          pltpu.VMEM_SHARED(shp(n_pad), jnp.float32),
          pltpu.VMEM(shp(window), jnp.float32),
          pltpu.VMEM(shp(window), jnp.float32),
          pltpu.VMEM((window,), jnp.int32),
          pltpu.VMEM((window,), jnp.int32),
          pltpu.VMEM((window,), jnp.int32),
          pltpu.VMEM((window,), jnp.int32),
          pltpu.SemaphoreType.DMA,
          pltpu.SemaphoreType.DMA,
          pltpu.SemaphoreType.DMA,
          pltpu.SemaphoreType.DMA,
          pltpu.SemaphoreType.DMA,
          pltpu.SemaphoreType.DMA,
      ],
  )


_HI = lax.Precision.HIGHEST


def _tc_node1(degp_ref, x_ref, rep_ref, dis_ref, y1_ref):
  dis8 = lax.rsqrt(1.0 + degp_ref[0] + degp_ref[1])
  dis_ref[...] = dis8
  x_r = x_ref[...].astype(jnp.bfloat16).astype(jnp.float32)
  disr = jnp.dot(dis8, rep_ref[...], precision=_HI)
  y1_ref[...] = x_r * disr


def _tc_node2(accp_ref, y1_ref, dis_ref, rep_ref, b_ref, b1_ref, c_ref,
              y2_ref):
  dis8 = dis_ref[...]
  disr = jnp.dot(dis8, rep_ref[...], precision=_HI)
  p1 = disr * (accp_ref[0] + accp_ref[1] + y1_ref[...])
  h = jnp.dot(p1, b_ref[...], precision=_HI) + b1_ref[...]
  h = jnp.maximum(h, 0.0)
  h = h.astype(jnp.bfloat16).astype(jnp.float32)
  z = jnp.dot(h, c_ref[...], precision=_HI)
  y2_ref[...] = z * dis8


def _tc_node3(accp_ref, y2_ref, dis_ref, b2_ref, out_ref):
  out_ref[...] = dis_ref[...] * (accp_ref[0] + accp_ref[1] + y2_ref[...]) \
      + b2_ref[0]


def kernel(x, edge_index, W1, b1, W2, b2):
  n = x.shape[0]
  nf_in = x.shape[1]
  nf_hid = W1.shape[1]
  n_edges = edge_index.shape[1]
  window = 4000
  window_h = 4000
  window_r = 1000
  n_pad = 102400
  rows = n_pad // 128
  fp = 8
  g = n_pad // fp

  src = edge_index[0].astype(jnp.int32)
  dst = edge_index[1].astype(jnp.int32)
  W1r = jnp.zeros((fp, nf_hid), jnp.float32).at[:nf_in].set(
      W1.astype(jnp.bfloat16).astype(jnp.float32))
  W2r = W2.astype(jnp.bfloat16).astype(jnp.float32)
  eye8 = jnp.eye(fp, dtype=jnp.float32)
  rep_m = jnp.kron(eye8, jnp.ones((1, fp), jnp.float32))
  b_m = jnp.kron(eye8, W1r)
  c_m = jnp.kron(eye8, W2r)
  b1_t = jnp.tile(b1, fp).reshape(1, fp * nf_hid)
  zeros_n = jnp.zeros((n_pad,), jnp.float32)
  ones_w = jnp.ones((window_h,), jnp.float32)
  x_f = jnp.pad(x, ((0, n_pad - n), (0, fp - nf_in))).reshape(g, fp * fp)

  degp = _sc_hist(n_pad, n_edges, window_h)(dst, zeros_n, ones_w)

  gb = g // 4
  dis8, y1_f = pl.pallas_call(
      _tc_node1,
      grid=(4,),
      in_specs=[
          pl.BlockSpec((NC, gb, fp), lambda i: (0, i, 0)),
          pl.BlockSpec((gb, fp * fp), lambda i: (i, 0)),
          pl.BlockSpec((fp, fp * fp), lambda i: (0, 0)),
      ],
      out_specs=(
          pl.BlockSpec((gb, fp), lambda i: (i, 0)),
          pl.BlockSpec((gb, fp * fp), lambda i: (i, 0)),
      ),
      out_shape=(
          jax.ShapeDtypeStruct((g, fp), jnp.float32),
          jax.ShapeDtypeStruct((g, fp * fp), jnp.float32),
      ),
  )(degp.reshape(NC, g, fp), x_f, rep_m)

  zeros_nf = jnp.zeros((n_pad, fp), jnp.float32)
  acc1_rm = _sc_gs(n_pad, n_edges, window_r, fp)(
      src, dst, zeros_nf, y1_f.reshape(n_pad, fp))

  y2 = pl.pallas_call(
      _tc_node2,
      grid=(4,),
      in_specs=[
          pl.BlockSpec((NC, gb, fp * fp), lambda i: (0, i, 0)),
          pl.BlockSpec((gb, fp * fp), lambda i: (i, 0)),
          pl.BlockSpec((gb, fp), lambda i: (i, 0)),
          pl.BlockSpec((fp, fp * fp), lambda i: (0, 0)),
          pl.BlockSpec((fp * fp, fp * nf_hid), lambda i: (0, 0)),
          pl.BlockSpec((1, fp * nf_hid), lambda i: (0, 0)),
          pl.BlockSpec((fp * nf_hid, fp), lambda i: (0, 0)),
      ],
      out_specs=pl.BlockSpec((gb, fp), lambda i: (i, 0)),
      out_shape=jax.ShapeDtypeStruct((g, fp), jnp.float32),
  )(acc1_rm.reshape(NC, g, fp * fp), y1_f, dis8, rep_m, b_m, b1_t, c_m)

  acc2 = _sc_gs(n_pad, n_edges, window, 0)(
      src, dst, zeros_n, y2.reshape(n_pad))

  out = pl.pallas_call(
      _tc_node3,
      in_specs=[
          pl.BlockSpec(memory_space=pltpu.MemorySpace.VMEM),
          pl.BlockSpec(memory_space=pltpu.MemorySpace.VMEM),
          pl.BlockSpec(memory_space=pltpu.MemorySpace.VMEM),
          pl.BlockSpec(memory_space=pltpu.SMEM),
      ],
      out_shape=jax.ShapeDtypeStruct((rows, 128), jnp.float32),
  )(acc2.reshape(NC, rows, 128), y2.reshape(rows, 128), dis8.reshape(rows, 128), b2)

  return out.reshape(n_pad)[:n].reshape(n, 1)
